# baseline (speedup 1.0000x reference)
import functools

import jax
import jax.numpy as jnp
from jax import lax
from jax.experimental import pallas as pl
from jax.experimental.pallas import tpu as pltpu
from jax.experimental.pallas import tpu_sc as plsc

N_CAT = 26
VOCAB = 100000
D = 32
B = 16384
L = 16

NC, NS = 2, 16
NW = NC * NS
BPW = B // NW
BT_W = BPW // 128
DT = D // 8
TILE = 8 * 128
QROWS = N_CAT * VOCAB // 4
QPF = VOCAB // 4


def _body(x_hbm, tab_hbm, btile_hbm, out_hbm,
          wq_v, lb_v, rows_v, tile_v, btile_v, sem_g, sem_w):
    wid = lax.axis_index("s") * NC + lax.axis_index("c")
    b0 = wid * BPW

    pltpu.sync_copy(x_hbm.at[:, pl.ds(b0, BPW)], wq_v)
    pltpu.sync_copy(btile_hbm, btile_v)

    def split_idx(f, _):
        fbase = jnp.full((L,), QPF, jnp.int32) * f
        three = jnp.full((L,), 3, jnp.int32)
        for k in range(BPW // L):
            sl = pl.ds(k * L, L)
            x = wq_v[f, sl]
            lb_v[f, sl] = lax.shift_left(lax.bitwise_and(x, three), 5)
            wq_v[f, sl] = lax.shift_right_logical(x, 2) + fbase
        return 0
    lax.fori_loop(0, N_CAT, split_idx, 0)

    def gather(f, j, s):
        return pltpu.make_async_copy(
            tab_hbm.at[wq_v.at[f, pl.ds(j * 128, 128)]], rows_v.at[s],
            sem_g.at[s])

    def write(f, j, t):
        return pltpu.make_async_copy(
            tile_v.at[t], out_hbm.at[f, :, wid * BT_W + j, :], sem_w.at[t])

    for j in range(2):
        gather(0, j, j).start()

    iota = lax.iota(jnp.int32, L)

    def item(i52, _):
        for k in range(2):
            ii = i52 * 2 + k
            f = lax.shift_right_logical(ii, 2)
            j = lax.bitwise_and(ii, 3)
            s = k
            t = k
            gather(f, j, s).wait()

            @pl.when(ii >= 2)
            def _():
                write(f, j, t).wait()

            base = [iota + (g * L) for g in range(128 // L)]
            lbs = [lb_v[f, pl.ds(j * 128 + g * L, L)] for g in range(128 // L)]

            for d in range(D):
                dt, dsub = d // 8, d % 8
                bias = btile_v[f, d, :]
                dvec = jnp.full((L,), d, jnp.int32)
                for g in range(128 // L):
                    col = lbs[g] + dvec
                    v = plsc.load_gather(rows_v.at[s], [base[g], col])
                    tile_v[t, dt, pl.ds(dsub * 128 + g * L, L)] = v + bias

            write(f, j, t).start()

            @pl.when(ii + 2 < N_CAT * BT_W)
            def _():
                f2 = lax.shift_right_logical(ii + 2, 2)
                j2 = lax.bitwise_and(ii + 2, 3)
                gather(f2, j2, s).start()
        return 0
    lax.fori_loop(0, N_CAT * BT_W // 2, item, 0)

    write(N_CAT - 1, BT_W - 2, 0).wait()
    write(N_CAT - 1, BT_W - 1, 1).wait()


def kernel(x_categ, tables, biases):
    xf = x_categ.T.reshape(N_CAT, B)
    tab = tables.reshape(QROWS, 128)
    btile = jnp.broadcast_to(biases[:, :, None], (N_CAT, D, L))

    mesh = plsc.VectorSubcoreMesh(core_axis_name="c", subcore_axis_name="s")
    out4 = pl.kernel(
        _body,
        mesh=mesh,
        out_type=jax.ShapeDtypeStruct((N_CAT, DT, B // 128, TILE),
                                      jnp.float32),
        compiler_params=pltpu.CompilerParams(use_tc_tiling_on_sc=False,
                                             needs_layout_passes=False),
        scratch_types=[
            pltpu.VMEM((N_CAT, BPW), jnp.int32),
            pltpu.VMEM((N_CAT, BPW), jnp.int32),
            pltpu.VMEM((2, 128, 128), jnp.float32),
            pltpu.VMEM((2, DT, TILE), jnp.float32),
            pltpu.VMEM((N_CAT, D, L), jnp.float32),
            pltpu.SemaphoreType.DMA((2,)),
            pltpu.SemaphoreType.DMA((2,)),
        ],
    )(xf, tab, btile)
    out5 = out4.reshape(N_CAT, DT, B // 128, 8, 128)
    return out5.transpose(2, 4, 0, 1, 3).reshape(B, N_CAT, D)


# speedup vs baseline: 1.1456x; 1.1456x over previous
import functools

import jax
import jax.numpy as jnp
from jax import lax
from jax.experimental import pallas as pl
from jax.experimental.pallas import tpu as pltpu
from jax.experimental.pallas import tpu_sc as plsc

N_CAT = 26
VOCAB = 100000
D = 32
B = 16384
L = 16

NC, NS = 2, 16
NW = NC * NS
BPW = B // NW
BT_W = BPW // 128
DT = D // 8
TILE = 8 * 128


def _body(x_hbm, tab_hbm, btile_hbm, out_hbm,
          xi_v, rows_v, tile_v, btile_v, sem_g, sem_w):
    wid = lax.axis_index("s") * NC + lax.axis_index("c")
    b0 = wid * BPW

    pltpu.sync_copy(x_hbm.at[:, pl.ds(b0, BPW)], xi_v)
    pltpu.sync_copy(btile_hbm, btile_v)

    def gather(f, j, s):
        return pltpu.make_async_copy(
            tab_hbm.at[f].at[xi_v.at[f, pl.ds(j * 128, 128)]], rows_v.at[s],
            sem_g.at[s])

    def write(f, j, t):
        return pltpu.make_async_copy(
            tile_v.at[t], out_hbm.at[f, :, wid * BT_W + j, :], sem_w.at[t])

    for j in range(2):
        gather(0, j, j).start()

    iota = lax.iota(jnp.int32, L)
    base = [iota + (g * L) for g in range(128 // L)]
    dvec = [jnp.full((L,), d, jnp.int32) for d in range(D)]

    def item(i52, _):
        for k in range(2):
            ii = i52 * 2 + k
            f = lax.shift_right_logical(ii, 2)
            j = lax.bitwise_and(ii, 3)
            s = k
            t = k
            gather(f, j, s).wait()

            @pl.when(ii >= 2)
            def _():
                write(f, j, t).wait()

            for half in range(2):
                ds0 = half * (D // 2)
                bias16 = [btile_v[f, ds0 + i, :] for i in range(D // 2)]
                for g in range(128 // L):
                    vals = []
                    for i in range(D // 2):
                        vals.append(plsc.load_gather(
                            rows_v.at[s], [base[g], dvec[ds0 + i]]))
                    for i in range(D // 2):
                        d = ds0 + i
                        dt, dsub = d // 8, d % 8
                        tile_v[t, dt, pl.ds(dsub * 128 + g * L, L)] = (
                            vals[i] + bias16[i])

            write(f, j, t).start()

            @pl.when(ii + 2 < N_CAT * BT_W)
            def _():
                f2 = lax.shift_right_logical(ii + 2, 2)
                j2 = lax.bitwise_and(ii + 2, 3)
                gather(f2, j2, s).start()
        return 0
    lax.fori_loop(0, N_CAT * BT_W // 2, item, 0)

    write(N_CAT - 1, BT_W - 2, 0).wait()
    write(N_CAT - 1, BT_W - 1, 1).wait()


def kernel(x_categ, tables, biases):
    xf = x_categ.T.reshape(N_CAT, B)
    btile = jnp.broadcast_to(biases[:, :, None], (N_CAT, D, L))

    mesh = plsc.VectorSubcoreMesh(core_axis_name="c", subcore_axis_name="s")
    out4 = pl.kernel(
        _body,
        mesh=mesh,
        out_type=jax.ShapeDtypeStruct((N_CAT, DT, B // 128, TILE),
                                      jnp.float32),
        compiler_params=pltpu.CompilerParams(use_tc_tiling_on_sc=False,
                                             needs_layout_passes=False),
        scratch_types=[
            pltpu.VMEM((N_CAT, BPW), jnp.int32),
            pltpu.VMEM((2, 128, D), jnp.float32),
            pltpu.VMEM((2, DT, TILE), jnp.float32),
            pltpu.VMEM((N_CAT, D, L), jnp.float32),
            pltpu.SemaphoreType.DMA((2,)),
            pltpu.SemaphoreType.DMA((2,)),
        ],
    )(xf, tables, btile)
    out5 = out4.reshape(N_CAT, DT, B // 128, 8, 128)
    return out5.transpose(2, 4, 0, 1, 3).reshape(B, N_CAT, D)
